# baseline (speedup 1.0000x reference)
import functools

import jax
import jax.numpy as jnp
from jax import lax
from jax.experimental import pallas as pl
from jax.experimental.pallas import tpu as pltpu
from jax.experimental.pallas import tpu_sc as plsc

_N_DIM = 128
_N_GROUP = 1000
_BATCH = 16384

_NC = 2
_NS = 16
_NW = _NC * _NS
_B_PER_W = _BATCH // _NW
_HALF = _B_PER_W // 2

_mesh = plsc.VectorSubcoreMesh(core_axis_name="c", subcore_axis_name="s")


@functools.partial(
    pl.kernel,
    mesh=_mesh,
    out_type=jax.ShapeDtypeStruct((_BATCH, _N_DIM), jnp.float32),
    scratch_types=[
        pltpu.VMEM((_HALF,), jnp.int32),
        pltpu.VMEM((_HALF,), jnp.int32),
        pltpu.VMEM((_B_PER_W, _N_DIM), jnp.float32),
        pltpu.SemaphoreType.DMA((2,)),
        pltpu.SemaphoreType.DMA,
    ],
)
def _gather_rows(idx_hbm, table_hbm, out_hbm, idx_a, idx_b, rows_v, gsems, osem):
    wid = lax.axis_index("s") * _NC + lax.axis_index("c")
    base = wid * _B_PER_W
    pltpu.sync_copy(idx_hbm.at[pl.ds(base, _HALF)], idx_a)
    pltpu.sync_copy(idx_hbm.at[pl.ds(base + _HALF, _HALF)], idx_b)
    g0 = pltpu.async_copy(
        table_hbm.at[idx_a], rows_v.at[pl.ds(0, _HALF)], gsems.at[0]
    )
    g1 = pltpu.async_copy(
        table_hbm.at[idx_b], rows_v.at[pl.ds(_HALF, _HALF)], gsems.at[1]
    )
    g0.wait()
    w0 = pltpu.async_copy(
        rows_v.at[pl.ds(0, _HALF)], out_hbm.at[pl.ds(base, _HALF)], osem
    )
    g1.wait()
    w1 = pltpu.async_copy(
        rows_v.at[pl.ds(_HALF, _HALF)],
        out_hbm.at[pl.ds(base + _HALF, _HALF)],
        osem,
    )
    w0.wait()
    w1.wait()


def kernel(inputs, w_all):
    idx = inputs.astype(jnp.int32)
    out = _gather_rows(idx, w_all)
    return out[:, :, None]


# speedup vs baseline: 1.0661x; 1.0661x over previous
import functools

import jax
import jax.numpy as jnp
from jax import lax
from jax.experimental import pallas as pl
from jax.experimental.pallas import tpu as pltpu
from jax.experimental.pallas import tpu_sc as plsc

_N_DIM = 128
_N_GROUP = 1000
_BATCH = 16384

_NC = 2
_NS = 16
_NW = _NC * _NS
_B_PER_W = _BATCH // _NW
_HALF = _B_PER_W // 2

_mesh = plsc.VectorSubcoreMesh(core_axis_name="c", subcore_axis_name="s")


@functools.partial(
    pl.kernel,
    mesh=_mesh,
    out_type=jax.ShapeDtypeStruct((_BATCH, _N_DIM), jnp.float32),
    scratch_types=[
        pltpu.VMEM((_B_PER_W,), jnp.int32),
        pltpu.VMEM((_B_PER_W, _N_DIM), jnp.float32),
        pltpu.SemaphoreType.DMA,
    ],
)
def _gather_rows(idx_hbm, table_hbm, out_hbm, idx_v, rows_v, sem):
    wid = lax.axis_index("s") * _NC + lax.axis_index("c")
    base = wid * _B_PER_W
    pltpu.sync_copy(idx_hbm.at[pl.ds(base, _B_PER_W)], idx_v)
    pltpu.async_copy(table_hbm.at[idx_v], rows_v, sem).wait()
    pltpu.sync_copy(rows_v, out_hbm.at[pl.ds(base, _B_PER_W)])


def kernel(inputs, w_all):
    idx = inputs.astype(jnp.int32)
    out = _gather_rows(idx, w_all)
    return out[:, :, None]
